# Initial kernel scaffold; baseline (speedup 1.0000x reference)
import jax
import jax.numpy as jnp
from jax.experimental import pallas as pl


def kernel(x):
    raise NotImplementedError("write your pallas kernel here")


# speedup vs baseline: 13.9860x; 13.9860x over previous
import numpy as np
import jax
import jax.numpy as jnp
from jax import lax
from jax.experimental import pallas as pl
from jax.experimental.pallas import tpu as pltpu
from jax.experimental.pallas import tpu_sc as plsc

K = 16
N_TOTAL = 1048576
NC, NS = 2, 16
NW = NC * NS
PER_W = N_TOTAL // NW
CHUNK = 2048
NCHUNKS = PER_W // CHUNK
VPC = CHUNK // K


def _cubic_C() -> np.ndarray:
    m = np.zeros((K - 2, K - 2))
    t = np.zeros((K - 2, K))
    for i in range(K - 2):
        m[i, i] = 2.0 / 3.0
        if i + 1 < K - 2:
            m[i, i + 1] = 1.0 / 6.0
            m[i + 1, i] = 1.0 / 6.0
        t[i, i], t[i, i + 1], t[i, i + 2] = 1.0, -2.0, 1.0
    c = np.vstack([np.zeros((1, K)), np.linalg.solve(m, t), np.zeros((1, K))])
    return np.ascontiguousarray(c.reshape(-1).astype(np.float32))


_C_FLAT = _cubic_C()

_MESH = plsc.VectorSubcoreMesh(core_axis_name="c", subcore_axis_name="s")


def _worker_id():
    return lax.axis_index("s") * NC + lax.axis_index("c")


def _minmax_body(x_hbm, part_hbm, xbuf, mnbuf, mxbuf):
    w = _worker_id()
    base = w * PER_W
    pltpu.sync_copy(x_hbm.at[pl.ds(base, CHUNK)], xbuf)
    mn0 = xbuf[pl.ds(0, K)]

    def chunk_step(k, carry):
        mnv, mxv = carry
        pltpu.sync_copy(x_hbm.at[pl.ds(base + k * CHUNK, CHUNK)], xbuf)

        def vstep(i, c):
            a, b = c
            xv = xbuf[pl.ds(i * K, K)]
            return jnp.minimum(a, xv), jnp.maximum(b, xv)

        return lax.fori_loop(0, VPC, vstep, (mnv, mxv))

    def vstep0(i, c):
        a, b = c
        xv = xbuf[pl.ds(i * K, K)]
        return jnp.minimum(a, xv), jnp.maximum(b, xv)

    mnv, mxv = lax.fori_loop(0, VPC, vstep0, (mn0, mn0))
    mnv, mxv = lax.fori_loop(1, NCHUNKS, chunk_step, (mnv, mxv))
    mnbuf[...] = mnv
    mxbuf[...] = mxv
    pltpu.sync_copy(mnbuf, part_hbm.at[pl.ds(w * K, K)])
    pltpu.sync_copy(mxbuf, part_hbm.at[pl.ds((NW + w) * K, K)])


def _expand_body(x_hbm, part_hbm, c_hbm, out_hbm, xbuf, pbuf, kbuf, fbuf, obuf):
    w = _worker_id()

    pltpu.sync_copy(part_hbm, pbuf)
    mnv = pbuf[pl.ds(0, K)]
    mxv = pbuf[pl.ds(NW * K, K)]
    for i in range(1, NW):
        mnv = jnp.minimum(mnv, pbuf[pl.ds(i * K, K)])
        mxv = jnp.maximum(mxv, pbuf[pl.ds((NW + i) * K, K)])

    lanei = lax.iota(jnp.int32, K)
    for sh in (8, 4, 2, 1):
        kbuf[...] = mnv
        mnv = jnp.minimum(mnv, plsc.load_gather(kbuf, [lanei ^ sh]))
        kbuf[...] = mxv
        mxv = jnp.maximum(mxv, plsc.load_gather(kbuf, [lanei ^ sh]))

    lanef = lanei.astype(jnp.float32)
    sv = lanef / 15.0
    xkv = mnv * (1.0 - sv) + mxv * sv
    xkv = jnp.where(lanei == K - 1, mxv, xkv)
    kbuf[...] = xkv

    stepv = (mxv - mnv) / 15.0
    invhb = 1.0 / stepv
    inv_h2v = invhb * invhb
    pltpu.sync_copy(c_hbm, fbuf)
    for s in range(K):
        fbuf[pl.ds(s * K, K)] = fbuf[pl.ds(s * K, K)] * inv_h2v

    mnb = mnv
    lane16 = lanei * K
    sixth = jnp.float32(1.0 / 6.0)

    base = w * PER_W

    def vbody(i, _):
        xv = xbuf[pl.ds(i * K, K)]
        t = (xv - mnb) * invhb
        j0 = jnp.clip(t.astype(jnp.int32) + 1, 1, K - 2)
        xlo = plsc.load_gather(kbuf, [j0 - 1])
        xhi = plsc.load_gather(kbuf, [j0])
        j = j0 - (xv <= xlo).astype(jnp.int32) + (xhi < xv).astype(jnp.int32)
        jm = jnp.where(j == 0, K - 1, j - 1)
        xj = plsc.load_gather(kbuf, [jm])
        xj1 = plsc.load_gather(kbuf, [j])
        he = xj1 - xj
        inv_he = 1.0 / he
        d1 = xj1 - xv
        d0 = xv - xj
        a_jm = d1 * inv_he
        a_jp = d0 * inv_he
        c_jm = (d1 * d1 * d1 * inv_he - he * d1) * sixth
        c_jp = (d0 * d0 * d0 * inv_he - he * d0) * sixth
        jm16 = jm * K
        j16 = j * K
        obase = i * (K * K) + lane16
        for c in range(K):
            fm = plsc.load_gather(fbuf, [jm16 + c])
            fp = plsc.load_gather(fbuf, [j16 + c])
            plsc.store_scatter(obuf, [obase + c], c_jm * fm + c_jp * fp)
        plsc.addupdate_scatter(obuf, [obase + jm], a_jm)
        plsc.addupdate_scatter(obuf, [obase + j], a_jp)
        return 0

    for k in range(NCHUNKS):
        cbase = base + k * CHUNK
        pltpu.sync_copy(x_hbm.at[pl.ds(cbase, CHUNK)], xbuf)
        lax.fori_loop(0, VPC, vbody, 0)
        pltpu.sync_copy(obuf, out_hbm.at[pl.ds(cbase * K, CHUNK * K)])


def kernel(x):
    parts = pl.kernel(
        _minmax_body,
        out_type=jax.ShapeDtypeStruct((2 * NW * K,), jnp.float32),
        mesh=_MESH,
        scratch_types=[
            pltpu.MemorySpace.VMEM((CHUNK,), jnp.float32),
            pltpu.MemorySpace.VMEM((K,), jnp.float32),
            pltpu.MemorySpace.VMEM((K,), jnp.float32),
        ],
        compiler_params=pltpu.CompilerParams(needs_layout_passes=False),
        name="cubic_minmax_sc",
    )(x)

    c_flat = jnp.asarray(_C_FLAT)
    out_flat = pl.kernel(
        _expand_body,
        out_type=jax.ShapeDtypeStruct((N_TOTAL * K,), jnp.float32),
        mesh=_MESH,
        scratch_types=[
            pltpu.MemorySpace.VMEM((CHUNK,), jnp.float32),
            pltpu.MemorySpace.VMEM((2 * NW * K,), jnp.float32),
            pltpu.MemorySpace.VMEM((K,), jnp.float32),
            pltpu.MemorySpace.VMEM((K * K,), jnp.float32),
            pltpu.MemorySpace.VMEM((CHUNK * K,), jnp.float32),
        ],
        compiler_params=pltpu.CompilerParams(needs_layout_passes=False),
        name="cubic_expand_sc",
    )(x, parts, c_flat)

    return out_flat.reshape(N_TOTAL, K)


# speedup vs baseline: 18.5422x; 1.3258x over previous
import numpy as np
import jax
import jax.numpy as jnp
from jax import lax
from jax.experimental import pallas as pl
from jax.experimental.pallas import tpu as pltpu
from jax.experimental.pallas import tpu_sc as plsc

K = 16
N_TOTAL = 1048576
NC, NS = 2, 16
NW = NC * NS
PER_W = N_TOTAL // NW
CHUNK = 2048
NCHUNKS = PER_W // CHUNK
VPC = CHUNK // K


def _cubic_C() -> np.ndarray:
    m = np.zeros((K - 2, K - 2))
    t = np.zeros((K - 2, K))
    for i in range(K - 2):
        m[i, i] = 2.0 / 3.0
        if i + 1 < K - 2:
            m[i, i + 1] = 1.0 / 6.0
            m[i + 1, i] = 1.0 / 6.0
        t[i, i], t[i, i + 1], t[i, i + 2] = 1.0, -2.0, 1.0
    c = np.vstack([np.zeros((1, K)), np.linalg.solve(m, t), np.zeros((1, K))])
    return np.ascontiguousarray(c.reshape(-1).astype(np.float32))


_C_FLAT = _cubic_C()

_MESH = plsc.VectorSubcoreMesh(core_axis_name="c", subcore_axis_name="s")


def _worker_id():
    return lax.axis_index("s") * NC + lax.axis_index("c")


def _minmax_body(x_hbm, part_hbm, xbuf, mnbuf, mxbuf):
    w = _worker_id()
    base = w * PER_W
    pltpu.sync_copy(x_hbm.at[pl.ds(base, CHUNK)], xbuf)
    mn0 = xbuf[pl.ds(0, K)]

    def chunk_step(k, carry):
        mnv, mxv = carry
        pltpu.sync_copy(x_hbm.at[pl.ds(base + k * CHUNK, CHUNK)], xbuf)

        def vstep(i, c):
            a, b = c
            xv = xbuf[pl.ds(i * K, K)]
            return jnp.minimum(a, xv), jnp.maximum(b, xv)

        return lax.fori_loop(0, VPC, vstep, (mnv, mxv))

    def vstep0(i, c):
        a, b = c
        xv = xbuf[pl.ds(i * K, K)]
        return jnp.minimum(a, xv), jnp.maximum(b, xv)

    mnv, mxv = lax.fori_loop(0, VPC, vstep0, (mn0, mn0))
    mnv, mxv = lax.fori_loop(1, NCHUNKS, chunk_step, (mnv, mxv))
    mnbuf[...] = mnv
    mxbuf[...] = mxv
    pltpu.sync_copy(mnbuf, part_hbm.at[pl.ds(w * K, K)])
    pltpu.sync_copy(mxbuf, part_hbm.at[pl.ds((NW + w) * K, K)])


def _expand_body(x_hbm, part_hbm, c_hbm, out_hbm, xbuf, pbuf, kbuf, fbuf, obuf):
    w = _worker_id()

    pltpu.sync_copy(part_hbm, pbuf)
    mnv = pbuf[pl.ds(0, K)]
    mxv = pbuf[pl.ds(NW * K, K)]
    for i in range(1, NW):
        mnv = jnp.minimum(mnv, pbuf[pl.ds(i * K, K)])
        mxv = jnp.maximum(mxv, pbuf[pl.ds((NW + i) * K, K)])

    lanei = lax.iota(jnp.int32, K)
    for sh in (8, 4, 2, 1):
        kbuf[...] = mnv
        mnv = jnp.minimum(mnv, plsc.load_gather(kbuf, [lanei ^ sh]))
        kbuf[...] = mxv
        mxv = jnp.maximum(mxv, plsc.load_gather(kbuf, [lanei ^ sh]))

    lanef = lanei.astype(jnp.float32)
    sv = lanef / 15.0
    xkv = mnv * (1.0 - sv) + mxv * sv
    xkv = jnp.where(lanei == K - 1, mxv, xkv)
    kbuf[...] = xkv

    stepv = (mxv - mnv) / 15.0
    invhb = 1.0 / stepv
    inv_h2v = invhb * invhb
    pltpu.sync_copy(c_hbm, fbuf)
    for s in range(K):
        fbuf[pl.ds(s * K, K)] = fbuf[pl.ds(s * K, K)] * inv_h2v

    mnb = mnv
    lane16 = lanei * K
    sixth = jnp.float32(1.0 / 6.0)

    base = w * PER_W

    def vbody(i, _):
        xv = xbuf[pl.ds(i * K, K)]
        t = (xv - mnb) * invhb
        j0 = jnp.clip(t.astype(jnp.int32) + 1, 1, K - 2)
        xlo = plsc.load_gather(kbuf, [j0 - 1])
        xhi = plsc.load_gather(kbuf, [j0])
        j = j0 - (xv <= xlo).astype(jnp.int32) + (xhi < xv).astype(jnp.int32)
        jm = jnp.where(j == 0, K - 1, j - 1)
        xj = plsc.load_gather(kbuf, [jm])
        xj1 = plsc.load_gather(kbuf, [j])
        he = xj1 - xj
        inv_he = 1.0 / he
        d1 = xj1 - xv
        d0 = xv - xj
        a_jm = d1 * inv_he
        a_jp = d0 * inv_he
        c_jm = (d1 * d1 * d1 * inv_he - he * d1) * sixth
        c_jp = (d0 * d0 * d0 * inv_he - he * d0) * sixth
        jm16 = jm * K
        j16 = j * K
        obase = i * (K * K) + lane16
        for c in range(K):
            col = lanei ^ c
            fm = plsc.load_gather(fbuf, [jm16 + col])
            fp = plsc.load_gather(fbuf, [j16 + col])
            plsc.store_scatter(obuf, [obase + col], c_jm * fm + c_jp * fp)
        plsc.addupdate_scatter(obuf, [obase + jm], a_jm)
        plsc.addupdate_scatter(obuf, [obase + j], a_jp)
        return 0

    for k in range(NCHUNKS):
        cbase = base + k * CHUNK
        pltpu.sync_copy(x_hbm.at[pl.ds(cbase, CHUNK)], xbuf)
        lax.fori_loop(0, VPC, vbody, 0)
        pltpu.sync_copy(obuf, out_hbm.at[pl.ds(cbase * K, CHUNK * K)])


def kernel(x):
    parts = pl.kernel(
        _minmax_body,
        out_type=jax.ShapeDtypeStruct((2 * NW * K,), jnp.float32),
        mesh=_MESH,
        scratch_types=[
            pltpu.MemorySpace.VMEM((CHUNK,), jnp.float32),
            pltpu.MemorySpace.VMEM((K,), jnp.float32),
            pltpu.MemorySpace.VMEM((K,), jnp.float32),
        ],
        compiler_params=pltpu.CompilerParams(needs_layout_passes=False),
        name="cubic_minmax_sc",
    )(x)

    c_flat = jnp.asarray(_C_FLAT)
    out_flat = pl.kernel(
        _expand_body,
        out_type=jax.ShapeDtypeStruct((N_TOTAL * K,), jnp.float32),
        mesh=_MESH,
        scratch_types=[
            pltpu.MemorySpace.VMEM((CHUNK,), jnp.float32),
            pltpu.MemorySpace.VMEM((2 * NW * K,), jnp.float32),
            pltpu.MemorySpace.VMEM((K,), jnp.float32),
            pltpu.MemorySpace.VMEM((K * K,), jnp.float32),
            pltpu.MemorySpace.VMEM((CHUNK * K,), jnp.float32),
        ],
        compiler_params=pltpu.CompilerParams(needs_layout_passes=False),
        name="cubic_expand_sc",
    )(x, parts, c_flat)

    return out_flat.reshape(N_TOTAL, K)


# speedup vs baseline: 19.2941x; 1.0405x over previous
import numpy as np
import jax
import jax.numpy as jnp
from jax import lax
from jax.experimental import pallas as pl
from jax.experimental.pallas import tpu as pltpu
from jax.experimental.pallas import tpu_sc as plsc

K = 16
N_TOTAL = 1048576
NC, NS = 2, 16
NW = NC * NS
PER_W = N_TOTAL // NW
CHUNK = 2048
NCHUNKS = PER_W // CHUNK
VPC = CHUNK // K


def _cubic_C() -> np.ndarray:
    m = np.zeros((K - 2, K - 2))
    t = np.zeros((K - 2, K))
    for i in range(K - 2):
        m[i, i] = 2.0 / 3.0
        if i + 1 < K - 2:
            m[i, i + 1] = 1.0 / 6.0
            m[i + 1, i] = 1.0 / 6.0
        t[i, i], t[i, i + 1], t[i, i + 2] = 1.0, -2.0, 1.0
    c = np.vstack([np.zeros((1, K)), np.linalg.solve(m, t), np.zeros((1, K))])
    return np.ascontiguousarray(c.reshape(-1).astype(np.float32))


_C_FLAT = _cubic_C()

_MESH = plsc.VectorSubcoreMesh(core_axis_name="c", subcore_axis_name="s")


def _worker_id():
    return lax.axis_index("s") * NC + lax.axis_index("c")


def _minmax_body(x_hbm, part_hbm, xbuf, mnbuf, mxbuf):
    w = _worker_id()
    base = w * PER_W
    pltpu.sync_copy(x_hbm.at[pl.ds(base, CHUNK)], xbuf)
    mn0 = xbuf[pl.ds(0, K)]

    def chunk_step(k, carry):
        mnv, mxv = carry
        pltpu.sync_copy(x_hbm.at[pl.ds(base + k * CHUNK, CHUNK)], xbuf)

        def vstep(i, c):
            a, b = c
            xv = xbuf[pl.ds(i * K, K)]
            return jnp.minimum(a, xv), jnp.maximum(b, xv)

        return lax.fori_loop(0, VPC, vstep, (mnv, mxv))

    def vstep0(i, c):
        a, b = c
        xv = xbuf[pl.ds(i * K, K)]
        return jnp.minimum(a, xv), jnp.maximum(b, xv)

    mnv, mxv = lax.fori_loop(0, VPC, vstep0, (mn0, mn0))
    mnv, mxv = lax.fori_loop(1, NCHUNKS, chunk_step, (mnv, mxv))
    mnbuf[...] = mnv
    mxbuf[...] = mxv
    pltpu.sync_copy(mnbuf, part_hbm.at[pl.ds(w * K, K)])
    pltpu.sync_copy(mxbuf, part_hbm.at[pl.ds((NW + w) * K, K)])


def _expand_body(x_hbm, part_hbm, c_hbm, out_hbm, xbuf, pbuf, kbuf, fbuf, obuf,
                 isem0, isem1, osem0, osem1):
    w = _worker_id()
    isems = (isem0, isem1)
    osems = (osem0, osem1)

    pltpu.sync_copy(part_hbm, pbuf)
    mnv = pbuf[pl.ds(0, K)]
    mxv = pbuf[pl.ds(NW * K, K)]
    for i in range(1, NW):
        mnv = jnp.minimum(mnv, pbuf[pl.ds(i * K, K)])
        mxv = jnp.maximum(mxv, pbuf[pl.ds((NW + i) * K, K)])

    lanei = lax.iota(jnp.int32, K)
    for sh in (8, 4, 2, 1):
        kbuf[...] = mnv
        mnv = jnp.minimum(mnv, plsc.load_gather(kbuf, [lanei ^ sh]))
        kbuf[...] = mxv
        mxv = jnp.maximum(mxv, plsc.load_gather(kbuf, [lanei ^ sh]))

    lanef = lanei.astype(jnp.float32)
    sv = lanef / 15.0
    xkv = mnv * (1.0 - sv) + mxv * sv
    xkv = jnp.where(lanei == K - 1, mxv, xkv)
    kbuf[...] = xkv

    stepv = (mxv - mnv) / 15.0
    invhb = 1.0 / stepv
    inv_h2v = invhb * invhb
    pltpu.sync_copy(c_hbm, fbuf)
    for s in range(K):
        fbuf[pl.ds(s * K, K)] = fbuf[pl.ds(s * K, K)] * inv_h2v

    mnb = mnv
    lane16 = lanei * K
    sixth = jnp.float32(1.0 / 6.0)

    base = w * PER_W

    def make_vbody(xoff, ooff):
        def vbody(i, _):
            xv = xbuf[pl.ds(xoff + i * K, K)]
            t = (xv - mnb) * invhb
            j0 = jnp.clip(t.astype(jnp.int32) + 1, 1, K - 2)
            xlo = plsc.load_gather(kbuf, [j0 - 1])
            xhi = plsc.load_gather(kbuf, [j0])
            j = j0 - (xv <= xlo).astype(jnp.int32) + (xhi < xv).astype(jnp.int32)
            jm = jnp.where(j == 0, K - 1, j - 1)
            xj = plsc.load_gather(kbuf, [jm])
            xj1 = plsc.load_gather(kbuf, [j])
            he = xj1 - xj
            inv_he = 1.0 / he
            d1 = xj1 - xv
            d0 = xv - xj
            a_jm = d1 * inv_he
            a_jp = d0 * inv_he
            c_jm = (d1 * d1 * d1 * inv_he - he * d1) * sixth
            c_jp = (d0 * d0 * d0 * inv_he - he * d0) * sixth
            jm16 = jm * K
            j16 = j * K
            obase = ooff + i * (K * K) + lane16
            for c in range(K):
                col = lanei ^ c
                fm = plsc.load_gather(fbuf, [jm16 + col])
                fp = plsc.load_gather(fbuf, [j16 + col])
                plsc.store_scatter(obuf, [obase + col], c_jm * fm + c_jp * fp)
            plsc.addupdate_scatter(obuf, [obase + jm], a_jm)
            plsc.addupdate_scatter(obuf, [obase + j], a_jp)
            return 0

        return vbody

    ins = [None, None]
    outs = [None, None]
    ins[0] = pltpu.async_copy(
        x_hbm.at[pl.ds(base, CHUNK)], xbuf.at[pl.ds(0, CHUNK)], isems[0])
    for k in range(NCHUNKS):
        h = k & 1
        cbase = base + k * CHUNK
        if k + 1 < NCHUNKS:
            ins[1 - h] = pltpu.async_copy(
                x_hbm.at[pl.ds(cbase + CHUNK, CHUNK)],
                xbuf.at[pl.ds((1 - h) * CHUNK, CHUNK)], isems[1 - h])
        ins[h].wait()
        if outs[h] is not None:
            outs[h].wait()
        lax.fori_loop(0, VPC, make_vbody(h * CHUNK, h * CHUNK * K), 0)
        outs[h] = pltpu.async_copy(
            obuf.at[pl.ds(h * CHUNK * K, CHUNK * K)],
            out_hbm.at[pl.ds(cbase * K, CHUNK * K)], osems[h])
    outs[0].wait()
    outs[1].wait()


def kernel(x):
    parts = pl.kernel(
        _minmax_body,
        out_type=jax.ShapeDtypeStruct((2 * NW * K,), jnp.float32),
        mesh=_MESH,
        scratch_types=[
            pltpu.MemorySpace.VMEM((CHUNK,), jnp.float32),
            pltpu.MemorySpace.VMEM((K,), jnp.float32),
            pltpu.MemorySpace.VMEM((K,), jnp.float32),
        ],
        compiler_params=pltpu.CompilerParams(needs_layout_passes=False),
        name="cubic_minmax_sc",
    )(x)

    c_flat = jnp.asarray(_C_FLAT)
    out_flat = pl.kernel(
        _expand_body,
        out_type=jax.ShapeDtypeStruct((N_TOTAL * K,), jnp.float32),
        mesh=_MESH,
        scratch_types=[
            pltpu.MemorySpace.VMEM((2 * CHUNK,), jnp.float32),
            pltpu.MemorySpace.VMEM((2 * NW * K,), jnp.float32),
            pltpu.MemorySpace.VMEM((K,), jnp.float32),
            pltpu.MemorySpace.VMEM((K * K,), jnp.float32),
            pltpu.MemorySpace.VMEM((2 * CHUNK * K,), jnp.float32),
            pltpu.SemaphoreType.DMA,
            pltpu.SemaphoreType.DMA,
            pltpu.SemaphoreType.DMA,
            pltpu.SemaphoreType.DMA,
        ],
        compiler_params=pltpu.CompilerParams(needs_layout_passes=False),
        name="cubic_expand_sc",
    )(x, parts, c_flat)

    return out_flat.reshape(N_TOTAL, K)
